# Initial kernel scaffold; baseline (speedup 1.0000x reference)
import jax
import jax.numpy as jnp
from jax.experimental import pallas as pl


def kernel(continuous_actions, discrete_actions, W_cont, b_cont, ln_g, ln_b, tables, W_final, b_final):
    raise NotImplementedError("write your pallas kernel here")


# speedup vs baseline: 8.0134x; 8.0134x over previous
import functools

import jax
import jax.numpy as jnp
from jax import lax
from jax.experimental import pallas as pl
from jax.experimental.pallas import tpu as pltpu
from jax.experimental.pallas import tpu_sc as plsc

_NC = 2
_NS = 16
_IDXW = 128


def _sc_gather(tables_flat, idx2d, bn, d):
    nw = _NC * _NS
    per_w = bn // nw
    chunk = 1024
    n_chunks = per_w // chunk
    rpc = chunk // _IDXW
    per_w_rows = per_w // _IDXW

    mesh = plsc.VectorSubcoreMesh(core_axis_name="c", subcore_axis_name="s")

    @functools.partial(
        pl.kernel,
        out_type=jax.ShapeDtypeStruct((bn, d), jnp.float32),
        mesh=mesh,
        scratch_types=[
            pltpu.VMEM((rpc, _IDXW), jnp.int32),
            pltpu.VMEM((chunk, d), jnp.float32),
            pltpu.SemaphoreType.DMA,
        ],
        compiler_params=pltpu.CompilerParams(use_tc_tiling_on_sc=False),
    )
    def gather_kernel(table_hbm, idx_hbm, out_hbm, idx_v, rows_v, sem):
        wid = lax.axis_index("s") * _NC + lax.axis_index("c")
        row_base = wid * per_w_rows

        def body(i, carry):
            r0 = row_base + i * rpc
            pltpu.sync_copy(idx_hbm.at[pl.ds(r0, rpc)], idx_v)
            descs = []
            for j in range(rpc):
                descs.append(
                    pltpu.async_copy(
                        table_hbm.at[idx_v.at[j]],
                        rows_v.at[pl.ds(j * _IDXW, _IDXW)],
                        sem,
                    )
                )
            for dsc in descs:
                dsc.wait()
            pltpu.sync_copy(rows_v, out_hbm.at[pl.ds(r0 * _IDXW, chunk)])
            return carry

        lax.fori_loop(0, n_chunks, body, 0)

    return gather_kernel(tables_flat, idx2d)


def _tc_dense(x, w_cont, b_cont, ln_g, ln_b, disc2d, w_top, w_rest, b_final):
    b, cd = x.shape
    d = w_top.shape[1]
    nfd = w_rest.shape[0]
    blk = 1024
    grid = (b // blk,)

    def body(x_ref, wc, bc, g, bt, dref, wt, wr, bf, o_ref):
        h = jnp.dot(x_ref[...], wc[...], preferred_element_type=jnp.float32)
        h = h + bc[...]
        mu = jnp.mean(h, axis=-1, keepdims=True)
        var = jnp.mean((h - mu) ** 2, axis=-1, keepdims=True)
        hn = (h - mu) * lax.rsqrt(var + 1e-5) * g[...] + bt[...]
        cont = 0.5 * hn * (1.0 + lax.erf(hn * 0.7071067811865476))
        acc = jnp.dot(cont, wt[...], preferred_element_type=jnp.float32)
        acc = acc + jnp.dot(dref[...], wr[...], preferred_element_type=jnp.float32)
        o_ref[...] = acc + bf[...]

    return pl.pallas_call(
        body,
        grid=grid,
        in_specs=[
            pl.BlockSpec((blk, cd), lambda i: (i, 0)),
            pl.BlockSpec((cd, d), lambda i: (0, 0)),
            pl.BlockSpec((1, d), lambda i: (0, 0)),
            pl.BlockSpec((1, d), lambda i: (0, 0)),
            pl.BlockSpec((1, d), lambda i: (0, 0)),
            pl.BlockSpec((blk, nfd), lambda i: (i, 0)),
            pl.BlockSpec((d, d), lambda i: (0, 0)),
            pl.BlockSpec((nfd, d), lambda i: (0, 0)),
            pl.BlockSpec((1, d), lambda i: (0, 0)),
        ],
        out_specs=pl.BlockSpec((blk, d), lambda i: (i, 0)),
        out_shape=jax.ShapeDtypeStruct((b, d), jnp.float32),
        compiler_params=pltpu.CompilerParams(
            dimension_semantics=("arbitrary",),
        ),
    )(x, w_cont, b_cont, ln_g, ln_b, disc2d, w_top, w_rest, b_final)


def kernel(continuous_actions, discrete_actions, W_cont, b_cont, ln_g, ln_b,
           tables, W_final, b_final):
    b, cd = continuous_actions.shape
    nf = discrete_actions.shape[1]
    v, d = tables.shape[1], tables.shape[2]
    bn = b * nf

    offs = (jnp.arange(nf, dtype=jnp.int32) * v)[None, :]
    flat_idx = (discrete_actions.astype(jnp.int32) + offs).reshape(bn // _IDXW, _IDXW)
    tables_flat = tables.reshape(nf * v, d)

    disc = _sc_gather(tables_flat, flat_idx, bn, d)
    disc2d = disc.reshape(b, nf * d)

    out = _tc_dense(
        continuous_actions,
        W_cont,
        b_cont.reshape(1, d),
        ln_g.reshape(1, d),
        ln_b.reshape(1, d),
        disc2d,
        W_final[:d],
        W_final[d:],
        b_final.reshape(1, d),
    )
    return out


# speedup vs baseline: 17.6588x; 2.2036x over previous
import functools

import jax
import jax.numpy as jnp
from jax import lax
from jax.experimental import pallas as pl
from jax.experimental.pallas import tpu as pltpu
from jax.experimental.pallas import tpu_sc as plsc

_NC = 2
_NS = 16
_LANE = 128


def _sc_plane_gather(tab3, tail, idxT, nf, v, d, b):
    nplane = nf * d
    nw = _NC * _NS
    ppw = nplane // nw
    cb = 1024
    nchunk = b // cb
    vmain = (v // _LANE) * _LANE
    vtail = v - vmain

    mesh = plsc.VectorSubcoreMesh(core_axis_name="c", subcore_axis_name="s")

    @functools.partial(
        pl.kernel,
        out_type=jax.ShapeDtypeStruct((nplane, b), jnp.float32),
        mesh=mesh,
        scratch_types=[
            pltpu.VMEM((vmain,), jnp.float32),
            pltpu.VMEM((nplane * vtail,), jnp.float32),
            pltpu.VMEM((cb,), jnp.int32),
            pltpu.VMEM((cb,), jnp.float32),
            pltpu.SemaphoreType.DMA,
        ],
        compiler_params=pltpu.CompilerParams(
            use_tc_tiling_on_sc=True, needs_layout_passes=False),
    )
    def plane_gather(tab3_hbm, tail_hbm, idxT_hbm, out_hbm,
                     plane_v, tail_v, idx_v, val_v, sem):
        wid = lax.axis_index("s") * _NC + lax.axis_index("c")
        pltpu.sync_copy(tail_hbm, tail_v)

        def task(t, carry):
            p = wid * ppw + t
            f = p // d
            s = p // 8
            r = p % 8
            pltpu.sync_copy(tab3_hbm.at[s, r, pl.ds(0, vmain)], plane_v)

            def chunk(ci, c2):
                pltpu.sync_copy(idxT_hbm.at[f, pl.ds(ci * cb, cb)], idx_v)
                tbase = p * vtail - vmain

                def grp(gi, c3):
                    iv = idx_v[pl.ds(gi * 16, 16)]
                    main = plsc.load_gather(
                        plane_v, [jnp.minimum(iv, vmain - 1)])
                    tl = plsc.load_gather(
                        tail_v,
                        [jnp.clip(iv + tbase, 0, nplane * vtail - 1)])
                    val_v[pl.ds(gi * 16, 16)] = jnp.where(iv < vmain, main, tl)
                    return c3

                lax.fori_loop(0, cb // 16, grp, 0)
                pltpu.sync_copy(val_v, out_hbm.at[p, pl.ds(ci * cb, cb)])
                return c2

            lax.fori_loop(0, nchunk, chunk, 0)
            return carry

        lax.fori_loop(0, ppw, task, 0)

    return plane_gather(tab3, tail, idxT)


def _tc_dense_t(xT, wcT, b_cont, ln_g, ln_b, discT, wtT, wrT, b_final):
    cd, b = xT.shape
    d = wcT.shape[0]
    nfd = wrT.shape[1]
    nb = 2048
    grid = (b // nb,)

    def body(x_ref, wc, bc, g, bt, dref, wt, wr, bf, o_ref):
        h = jnp.dot(wc[...], x_ref[...], preferred_element_type=jnp.float32)
        h = h + bc[...]
        mu = jnp.mean(h, axis=0, keepdims=True)
        var = jnp.mean((h - mu) ** 2, axis=0, keepdims=True)
        hn = (h - mu) * lax.rsqrt(var + 1e-5) * g[...] + bt[...]
        cont = 0.5 * hn * (1.0 + lax.erf(hn * 0.7071067811865476))
        acc = jnp.dot(wt[...], cont, preferred_element_type=jnp.float32)
        acc = acc + jnp.dot(wr[...], dref[...],
                            preferred_element_type=jnp.float32)
        o_ref[...] = acc + bf[...]

    return pl.pallas_call(
        body,
        grid=grid,
        in_specs=[
            pl.BlockSpec((cd, nb), lambda i: (0, i)),
            pl.BlockSpec((d, cd), lambda i: (0, 0)),
            pl.BlockSpec((d, 1), lambda i: (0, 0)),
            pl.BlockSpec((d, 1), lambda i: (0, 0)),
            pl.BlockSpec((d, 1), lambda i: (0, 0)),
            pl.BlockSpec((nfd, nb), lambda i: (0, i)),
            pl.BlockSpec((d, d), lambda i: (0, 0)),
            pl.BlockSpec((d, nfd), lambda i: (0, 0)),
            pl.BlockSpec((d, 1), lambda i: (0, 0)),
        ],
        out_specs=pl.BlockSpec((d, nb), lambda i: (0, i)),
        out_shape=jax.ShapeDtypeStruct((d, b), jnp.float32),
        compiler_params=pltpu.CompilerParams(
            dimension_semantics=("arbitrary",),
        ),
    )(xT, wcT, b_cont, ln_g, ln_b, discT, wtT, wrT, b_final)


def kernel(continuous_actions, discrete_actions, W_cont, b_cont, ln_g, ln_b,
           tables, W_final, b_final):
    b, cd = continuous_actions.shape
    nf = discrete_actions.shape[1]
    v, d = tables.shape[1], tables.shape[2]
    nplane = nf * d
    vmain = (v // _LANE) * _LANE

    tab_t = jnp.transpose(tables, (0, 2, 1))
    tab3 = tab_t.reshape(nf * 4, 8, v)
    tail = tab_t.reshape(nplane, v)[:, vmain:].reshape(-1)
    idxT = discrete_actions.T.astype(jnp.int32)

    discT = _sc_plane_gather(tab3, tail, idxT, nf, v, d, b)

    wfT = W_final.T
    outT = _tc_dense_t(
        continuous_actions.T,
        W_cont.T,
        b_cont.reshape(d, 1),
        ln_g.reshape(d, 1),
        ln_b.reshape(d, 1),
        discT,
        wfT[:, :d],
        wfT[:, d:],
        b_final.reshape(d, 1),
    )
    return outT.T


# speedup vs baseline: 21.3801x; 1.2107x over previous
import functools

import jax
import jax.numpy as jnp
from jax import lax
from jax.experimental import pallas as pl
from jax.experimental.pallas import tpu as pltpu
from jax.experimental.pallas import tpu_sc as plsc

_NC = 2
_NS = 16
_LANE = 128


def _sc_plane_gather(tab3, tail, idxT, nf, v, d, b):
    nplane = nf * d
    nw = _NC * _NS
    ppw = nplane // nw
    cb = 1024
    nchunk = b // cb
    vmain = (v // _LANE) * _LANE
    vtail = v - vmain

    mesh = plsc.VectorSubcoreMesh(core_axis_name="c", subcore_axis_name="s")

    @functools.partial(
        pl.kernel,
        out_type=jax.ShapeDtypeStruct((nplane, b), jnp.float32),
        mesh=mesh,
        scratch_types=[
            pltpu.VMEM((vmain + _LANE,), jnp.float32),
            pltpu.VMEM((nplane * vtail,), jnp.float32),
            pltpu.VMEM((cb,), jnp.int32),
            pltpu.VMEM((cb,), jnp.float32),
            pltpu.SemaphoreType.DMA,
        ],
        compiler_params=pltpu.CompilerParams(
            use_tc_tiling_on_sc=True, needs_layout_passes=False),
    )
    def plane_gather(tab3_hbm, tail_hbm, idxT_hbm, out_hbm,
                     plane_v, tail_v, idx_v, val_v, sem):
        wid = lax.axis_index("s") * _NC + lax.axis_index("c")
        pltpu.sync_copy(tail_hbm, tail_v)

        def task(t, carry):
            p = wid * ppw + t
            f = p // d
            s = p // 8
            r = p % 8
            pltpu.sync_copy(tab3_hbm.at[s, r, pl.ds(0, vmain)],
                            plane_v.at[pl.ds(0, vmain)])
            for q in range(vtail // 16):
                plane_v[pl.ds(vmain + q * 16, 16)] = (
                    tail_v[pl.ds(p * vtail + q * 16, 16)])

            def chunk(ci, c2):
                pltpu.sync_copy(idxT_hbm.at[f, pl.ds(ci * cb, cb)], idx_v)

                @plsc.parallel_loop(0, cb // 16, 1, unroll=8)
                def grp(gi):
                    iv = idx_v[pl.ds(gi * 16, 16)]
                    val_v[pl.ds(gi * 16, 16)] = plsc.load_gather(
                        plane_v, [iv])
                pltpu.sync_copy(val_v, out_hbm.at[p, pl.ds(ci * cb, cb)])
                return c2

            lax.fori_loop(0, nchunk, chunk, 0)
            return carry

        lax.fori_loop(0, ppw, task, 0)

    return plane_gather(tab3, tail, idxT)


def _tc_dense_t(xT, wcT, b_cont, ln_g, ln_b, discT, wtT, wrT, b_final):
    cd, b = xT.shape
    d = wcT.shape[0]
    nfd = wrT.shape[1]
    nb = 2048
    grid = (b // nb,)

    def body(x_ref, wc, bc, g, bt, dref, wt, wr, bf, o_ref):
        h = jnp.dot(wc[...], x_ref[...], preferred_element_type=jnp.float32)
        h = h + bc[...]
        mu = jnp.mean(h, axis=0, keepdims=True)
        var = jnp.mean((h - mu) ** 2, axis=0, keepdims=True)
        hn = (h - mu) * lax.rsqrt(var + 1e-5) * g[...] + bt[...]
        cont = 0.5 * hn * (1.0 + lax.erf(hn * 0.7071067811865476))
        acc = jnp.dot(wt[...], cont, preferred_element_type=jnp.float32)
        acc = acc + jnp.dot(wr[...], dref[...],
                            preferred_element_type=jnp.float32)
        o_ref[...] = acc + bf[...]

    return pl.pallas_call(
        body,
        grid=grid,
        in_specs=[
            pl.BlockSpec((cd, nb), lambda i: (0, i)),
            pl.BlockSpec((d, cd), lambda i: (0, 0)),
            pl.BlockSpec((d, 1), lambda i: (0, 0)),
            pl.BlockSpec((d, 1), lambda i: (0, 0)),
            pl.BlockSpec((d, 1), lambda i: (0, 0)),
            pl.BlockSpec((nfd, nb), lambda i: (0, i)),
            pl.BlockSpec((d, d), lambda i: (0, 0)),
            pl.BlockSpec((d, nfd), lambda i: (0, 0)),
            pl.BlockSpec((d, 1), lambda i: (0, 0)),
        ],
        out_specs=pl.BlockSpec((d, nb), lambda i: (0, i)),
        out_shape=jax.ShapeDtypeStruct((d, b), jnp.float32),
        compiler_params=pltpu.CompilerParams(
            dimension_semantics=("arbitrary",),
        ),
    )(xT, wcT, b_cont, ln_g, ln_b, discT, wtT, wrT, b_final)


def kernel(continuous_actions, discrete_actions, W_cont, b_cont, ln_g, ln_b,
           tables, W_final, b_final):
    b, cd = continuous_actions.shape
    nf = discrete_actions.shape[1]
    v, d = tables.shape[1], tables.shape[2]
    nplane = nf * d
    vmain = (v // _LANE) * _LANE

    tab_t = jnp.transpose(tables, (0, 2, 1))
    tab3 = tab_t.reshape(nf * 4, 8, v)
    tail = tab_t.reshape(nplane, v)[:, vmain:].reshape(-1)
    idxT = discrete_actions.T.astype(jnp.int32)

    discT = _sc_plane_gather(tab3, tail, idxT, nf, v, d, b)

    wfT = W_final.T
    outT = _tc_dense_t(
        continuous_actions.T,
        W_cont.T,
        b_cont.reshape(d, 1),
        ln_g.reshape(d, 1),
        ln_b.reshape(d, 1),
        discT,
        wfT[:, :d],
        wfT[:, d:],
        b_final.reshape(d, 1),
    )
    return outT.T


# speedup vs baseline: 47.9068x; 2.2407x over previous
import functools

import jax
import jax.numpy as jnp
from jax import lax
from jax.experimental import pallas as pl
from jax.experimental.pallas import tpu as pltpu
from jax.experimental.pallas import tpu_sc as plsc

_NC = 2
_NS = 16
_LANE = 128


def _sc_plane_gather(tab3, tail, idxT, nf, v, d, b):
    nplane = nf * d
    nw = _NC * _NS
    ppw = nplane // nw
    cb = 4096
    nchunk = b // cb
    vmain = (v // _LANE) * _LANE
    vtail = v - vmain
    splits = (200 * _LANE, 200 * _LANE, 200 * _LANE, vmain - 600 * _LANE)

    mesh = plsc.VectorSubcoreMesh(core_axis_name="c", subcore_axis_name="s")

    @functools.partial(
        pl.kernel,
        out_type=jax.ShapeDtypeStruct((nplane, b), jnp.float32),
        mesh=mesh,
        scratch_types=[
            pltpu.VMEM((vmain + _LANE,), jnp.float32),
            pltpu.VMEM((b,), jnp.int32),
            pltpu.VMEM((2, cb), jnp.float32),
            pltpu.SemaphoreType.DMA,
            pltpu.SemaphoreType.DMA,
        ],
        compiler_params=pltpu.CompilerParams(
            use_tc_tiling_on_sc=True, needs_layout_passes=False),
    )
    def plane_gather(tab3_hbm, tail_hbm, idxT_hbm, out_hbm,
                     plane_v, idx_v, val_v, psem, osem):
        wid = lax.axis_index("s") * _NC + lax.axis_index("c")

        def task(t, carry):
            p = wid * ppw + t
            f = p // d
            s = p // 8
            r = p % 8
            descs = []
            off = 0
            for ln in splits:
                descs.append(pltpu.async_copy(
                    tab3_hbm.at[s, r, pl.ds(off, ln)],
                    plane_v.at[pl.ds(off, ln)], psem))
                off += ln
            tw = (p * vtail) // _LANE * _LANE
            toff = p * vtail - tw
            descs.append(pltpu.async_copy(
                tail_hbm.at[pl.ds(tw, _LANE)],
                plane_v.at[pl.ds(vmain, _LANE)], psem))

            @pl.when(jnp.logical_or(t == 0, p % d == 0))
            def _():
                pltpu.sync_copy(idxT_hbm.at[f], idx_v)

            for dsc in descs:
                dsc.wait()
            lo = plane_v[pl.ds(vmain + toff, 16)]
            hi = plane_v[pl.ds(vmain + toff + 16, 16)]
            plane_v[pl.ds(vmain, 16)] = lo
            plane_v[pl.ds(vmain + 16, 16)] = hi

            def chunk(ci, c2):
                buf = ci % 2

                @plsc.parallel_loop(0, cb // 16, 1, unroll=8)
                def grp(gi):
                    iv = idx_v[pl.ds(ci * cb + gi * 16, 16)]
                    val_v[buf, pl.ds(gi * 16, 16)] = plsc.load_gather(
                        plane_v, [iv])

                @pl.when(ci >= 2)
                def _():
                    pltpu.make_async_copy(
                        val_v.at[buf], out_hbm.at[p, pl.ds(0, cb)],
                        osem).wait()
                pltpu.async_copy(val_v.at[buf],
                                 out_hbm.at[p, pl.ds(ci * cb, cb)], osem)
                return c2

            lax.fori_loop(0, nchunk, chunk, 0)
            for _ in range(2):
                pltpu.make_async_copy(
                    val_v.at[0], out_hbm.at[p, pl.ds(0, cb)], osem).wait()
            return carry

        lax.fori_loop(0, ppw, task, 0)

    return plane_gather(tab3, tail, idxT)


def _tc_dense_t(xT, wcT, b_cont, ln_g, ln_b, discT, wtT, wrT, b_final):
    cd, b = xT.shape
    d = wcT.shape[0]
    nfd = wrT.shape[1]
    nb = 2048
    grid = (b // nb,)

    def body(x_ref, wc, bc, g, bt, dref, wt, wr, bf, o_ref):
        h = jnp.dot(wc[...], x_ref[...], preferred_element_type=jnp.float32)
        h = h + bc[...]
        mu = jnp.mean(h, axis=0, keepdims=True)
        var = jnp.mean((h - mu) ** 2, axis=0, keepdims=True)
        hn = (h - mu) * lax.rsqrt(var + 1e-5) * g[...] + bt[...]
        cont = 0.5 * hn * (1.0 + lax.erf(hn * 0.7071067811865476))
        acc = jnp.dot(wt[...], cont, preferred_element_type=jnp.float32)
        acc = acc + jnp.dot(wr[...], dref[...],
                            preferred_element_type=jnp.float32)
        o_ref[...] = acc + bf[...]

    return pl.pallas_call(
        body,
        grid=grid,
        in_specs=[
            pl.BlockSpec((cd, nb), lambda i: (0, i)),
            pl.BlockSpec((d, cd), lambda i: (0, 0)),
            pl.BlockSpec((d, 1), lambda i: (0, 0)),
            pl.BlockSpec((d, 1), lambda i: (0, 0)),
            pl.BlockSpec((d, 1), lambda i: (0, 0)),
            pl.BlockSpec((nfd, nb), lambda i: (0, i)),
            pl.BlockSpec((d, d), lambda i: (0, 0)),
            pl.BlockSpec((d, nfd), lambda i: (0, 0)),
            pl.BlockSpec((d, 1), lambda i: (0, 0)),
        ],
        out_specs=pl.BlockSpec((d, nb), lambda i: (0, i)),
        out_shape=jax.ShapeDtypeStruct((d, b), jnp.float32),
        compiler_params=pltpu.CompilerParams(
            dimension_semantics=("arbitrary",),
        ),
    )(xT, wcT, b_cont, ln_g, ln_b, discT, wtT, wrT, b_final)


def kernel(continuous_actions, discrete_actions, W_cont, b_cont, ln_g, ln_b,
           tables, W_final, b_final):
    b, cd = continuous_actions.shape
    nf = discrete_actions.shape[1]
    v, d = tables.shape[1], tables.shape[2]
    nplane = nf * d
    vmain = (v // _LANE) * _LANE

    tab_t = jnp.transpose(tables, (0, 2, 1))
    tab3 = tab_t.reshape(nf * 4, 8, v)
    tail = tab_t.reshape(nplane, v)[:, vmain:].reshape(-1)
    idxT = discrete_actions.T.astype(jnp.int32)

    discT = _sc_plane_gather(tab3, tail, idxT, nf, v, d, b)

    wfT = W_final.T
    outT = _tc_dense_t(
        continuous_actions.T,
        W_cont.T,
        b_cont.reshape(d, 1),
        ln_g.reshape(d, 1),
        ln_b.reshape(d, 1),
        discT,
        wfT[:, :d],
        wfT[:, d:],
        b_final.reshape(d, 1),
    )
    return outT.T
